# baseline (speedup 1.0000x reference)
import jax
import jax.numpy as jnp
from jax.experimental import pallas as pl
from jax.experimental.pallas import tpu as pltpu

BATCH = 1000
FEAT_ROWS = 8
FEAT_COLS = 1536
BB = 8


def _fma_body(ts_ref, ac_ref, x0_ref, noise_ref, out_ref):
    pid = pl.program_id(0)
    for i in range(BB):
        t = ts_ref[pid * BB + i]
        a = ac_ref[t]
        sa = jnp.sqrt(a)
        sb = jnp.sqrt(jnp.maximum(1.0 - a, 0.0))
        out_ref[i] = sa * x0_ref[i] + sb * noise_ref[i]


def kernel(x_0, timesteps, noise, alphas_cumprod):
    orig_shape = x_0.shape
    x0r = x_0.reshape(BATCH, FEAT_ROWS, FEAT_COLS)
    nzr = noise.reshape(BATCH, FEAT_ROWS, FEAT_COLS)
    ts = timesteps.astype(jnp.int32)
    ac = alphas_cumprod.astype(jnp.float32)

    grid = (BATCH // BB,)
    out = pl.pallas_call(
        _fma_body,
        grid=grid,
        in_specs=[
            pl.BlockSpec(memory_space=pltpu.SMEM),
            pl.BlockSpec(memory_space=pltpu.SMEM),
            pl.BlockSpec((BB, FEAT_ROWS, FEAT_COLS), lambda i: (i, 0, 0)),
            pl.BlockSpec((BB, FEAT_ROWS, FEAT_COLS), lambda i: (i, 0, 0)),
        ],
        out_specs=pl.BlockSpec((BB, FEAT_ROWS, FEAT_COLS), lambda i: (i, 0, 0)),
        out_shape=jax.ShapeDtypeStruct((BATCH, FEAT_ROWS, FEAT_COLS), jnp.float32),
        compiler_params=pltpu.CompilerParams(
            dimension_semantics=("arbitrary",),
        ),
    )(ts, ac, x0r, nzr)
    return out.reshape(orig_shape)


# speedup vs baseline: 1.1520x; 1.1520x over previous
import functools

import jax
import jax.numpy as jnp
from jax import lax
from jax.experimental import pallas as pl
from jax.experimental.pallas import tpu as pltpu
from jax.experimental.pallas import tpu_sc as plsc

BATCH = 1000
FEAT = 12288
BB = 40

_SC_INFO = plsc.get_sparse_core_info()
_NC = _SC_INFO.num_cores
_NS = _SC_INFO.num_subcores
_NW = _NC * _NS
PAD_B = 1024
_PER_W = PAD_B // _NW


def _sc_gather_body(table_hbm, idx_hbm, out_hbm, idx_v, rows_v, sem):
    wid = lax.axis_index("s") * _NC + lax.axis_index("c")
    base = wid * _PER_W
    pltpu.sync_copy(idx_hbm.at[pl.ds(base, _PER_W)], idx_v)
    pltpu.async_copy(table_hbm.at[idx_v], rows_v, sem).wait()
    pltpu.sync_copy(rows_v, out_hbm.at[pl.ds(base, _PER_W)])


_sc_gather = functools.partial(
    pl.kernel,
    out_type=jax.ShapeDtypeStruct((PAD_B,), jnp.float32),
    mesh=plsc.VectorSubcoreMesh(core_axis_name="c", subcore_axis_name="s"),
    scratch_types=[
        pltpu.VMEM((_PER_W,), jnp.int32),
        pltpu.VMEM((_PER_W,), jnp.float32),
        pltpu.SemaphoreType.DMA,
    ],
)(_sc_gather_body)


def _fma_body(ac_t_ref, x0_ref, noise_ref, out_ref):
    a = ac_t_ref[...]
    sa = jnp.sqrt(a)
    sb = jnp.sqrt(jnp.maximum(1.0 - a, 0.0))
    out_ref[...] = sa * x0_ref[...] + sb * noise_ref[...]


def kernel(x_0, timesteps, noise, alphas_cumprod):
    orig_shape = x_0.shape
    x0r = x_0.reshape(BATCH, FEAT)
    nzr = noise.reshape(BATCH, FEAT)
    ts = timesteps.astype(jnp.int32)
    ac = alphas_cumprod.astype(jnp.float32)

    ts_pad = jnp.pad(ts, (0, PAD_B - BATCH))
    ac_pad = jnp.pad(ac, (0, PAD_B - BATCH))
    ac_t = _sc_gather(ac_pad, ts_pad)[:BATCH].reshape(BATCH, 1)

    grid = (BATCH // BB,)
    out = pl.pallas_call(
        _fma_body,
        grid=grid,
        in_specs=[
            pl.BlockSpec((BB, 1), lambda i: (i, 0)),
            pl.BlockSpec((BB, FEAT), lambda i: (i, 0)),
            pl.BlockSpec((BB, FEAT), lambda i: (i, 0)),
        ],
        out_specs=pl.BlockSpec((BB, FEAT), lambda i: (i, 0)),
        out_shape=jax.ShapeDtypeStruct((BATCH, FEAT), jnp.float32),
        compiler_params=pltpu.CompilerParams(
            dimension_semantics=("parallel",),
        ),
    )(ac_t, x0r, nzr)
    return out.reshape(orig_shape)
